# baseline (speedup 1.0000x reference)
import functools

import jax
import jax.numpy as jnp
from jax import lax
from jax.experimental import pallas as pl
from jax.experimental.pallas import tpu as pltpu
from jax.experimental.pallas import tpu_sc as plsc

VOCAB = 1000000
D = 64
PAD = 0
BATCH = 4096
SEQ = 200
TOTAL = BATCH * SEQ
NW = 32
GROUP = 128
NG = TOTAL // (NW * GROUP)


def _emb_body(x_hbm, table_hbm, out_hbm, idx_v, rows_v, sem):
    wid = lax.axis_index("s") * 2 + lax.axis_index("c")
    row0 = wid * NG
    pltpu.sync_copy(x_hbm.at[pl.ds(row0, NG)], idx_v)

    def gbody(g, carry):
        pltpu.async_copy(table_hbm.at[idx_v.at[g]], rows_v, sem).wait()

        def fix(h, c2):
            iv = idx_v[g, pl.ds(h * 16, 16)]
            m = iv == PAD
            npad = plsc.all_reduce_population_count(m)

            @pl.when(lax.squeeze(lax.slice(npad, (0,), (1,)), (0,)) > 0)
            def _():
                r16 = h * 16 + lax.iota(jnp.int32, 16)
                z = jnp.zeros((16,), jnp.float32)
                for j in range(D):
                    plsc.store_scatter(
                        rows_v, [r16, jnp.full((16,), j, jnp.int32)], z,
                        mask=m)

            return c2

        lax.fori_loop(0, GROUP // 16, fix, 0)
        pltpu.sync_copy(rows_v, out_hbm.at[pl.ds((row0 + g) * GROUP, GROUP)])
        return carry

    lax.fori_loop(0, NG, gbody, 0)


@jax.jit
def kernel(x, table):
    xi = x.reshape(TOTAL // GROUP, GROUP).astype(jnp.int32)
    k = functools.partial(
        pl.kernel,
        mesh=plsc.VectorSubcoreMesh(core_axis_name="c", subcore_axis_name="s"),
        compiler_params=pltpu.CompilerParams(
            needs_layout_passes=False, use_tc_tiling_on_sc=False),
        out_type=jax.ShapeDtypeStruct((TOTAL, D), jnp.float32),
        scratch_types=[
            pltpu.VMEM((NG, GROUP), jnp.int32),
            pltpu.VMEM((GROUP, D), jnp.float32),
            pltpu.SemaphoreType.DMA,
        ],
    )(_emb_body)
    out = k(xi, table)
    return out.reshape(BATCH, SEQ, D)
